# baseline (speedup 1.0000x reference)
import dataclasses
import functools

import jax
import jax.numpy as jnp
from jax import lax
from jax.experimental import pallas as pl
from jax.experimental.pallas import tpu as pltpu
from jax.experimental.pallas import tpu_sc as plsc

N = 10000
D = 128
N_ACC = 10112
NR = N_ACC // D
ND = 80
CHUNK = 64
NC = 2
NS = 16
NW = NC * NS
NBUF = 4
IB = 20


def _sc_compiler_params():
  cp = pltpu.CompilerParams(use_tc_tiling_on_sc=False)
  if "needs_layout_passes" in pltpu.CompilerParams.__dataclass_fields__:
    cp = dataclasses.replace(cp, needs_layout_passes=False)
  return cp


def _make_sc_agg(K):
  mesh = plsc.VectorSubcoreMesh(core_axis_name="c", subcore_axis_name="s")
  nblk = K // IB

  @functools.partial(
      pl.kernel,
      mesh=mesh,
      out_type=(
          jax.ShapeDtypeStruct((NC, N_ACC, D), jnp.float32),
          jax.ShapeDtypeStruct((NC, ND, D), jnp.float32),
      ),
      scratch_types=[
          pltpu.VMEM_SHARED((N_ACC, D), jnp.float32),
          pltpu.VMEM_SHARED((ND, D), jnp.float32),
          pltpu.VMEM((2, IB, 2, CHUNK), jnp.int32),
          pltpu.VMEM((NBUF, CHUNK, D), jnp.float32),
          pltpu.VMEM((ND, D), jnp.float32),
          pltpu.VMEM((ND,), jnp.int32),
      ] + [pltpu.SemaphoreType.DMA] * (2 * NBUF + 2),
      compiler_params=_sc_compiler_params(),
  )
  def sc_agg(xp_hbm, idx_hbm, zeros_hbm, out_hbm, deg_hbm, acc_sh, deg_sh,
             idx_v, rows_v, deg_v, ridx_v, *sems):
    gsem = sems[:NBUF]
    ssem = sems[NBUF:2 * NBUF]
    isem = sems[2 * NBUF:]
    c = lax.axis_index("c")
    s = lax.axis_index("s")
    wid = c * NS + s
    zrows = N_ACC // NS

    pltpu.async_copy(idx_hbm.at[pl.ds(wid * K, IB)], idx_v.at[0], isem[0])

    pltpu.sync_copy(
        zeros_hbm.at[pl.ds(s * zrows, zrows)],
        acc_sh.at[pl.ds(s * zrows, zrows)],
    )

    ones16 = jnp.ones((16,), jnp.float32)
    zeros16 = jnp.zeros((16,), jnp.float32)

    @pl.loop(0, ND)
    def _(r):
      for i in range(D // 16):
        deg_v[r, pl.ds(i * 16, 16)] = zeros16

    for i in range(ND // 16):
      ridx_v[pl.ds(i * 16, 16)] = lax.iota(jnp.int32, 16) + 16 * i

    @pl.when(s == 0)
    def _():
      pltpu.sync_copy(deg_v, deg_sh)

    plsc.subcore_barrier()

    @pl.loop(0, nblk, step=2)
    def _(kb0):
      for h in range(2):
        kb = kb0 + h
        pltpu.make_async_copy(
            idx_hbm.at[pl.ds(wid * K + kb * IB, IB)], idx_v.at[h],
            isem[h]).wait()

        @pl.when(kb + 1 < nblk)
        def _():
          pltpu.async_copy(
              idx_hbm.at[pl.ds(wid * K + (kb + 1) * IB, IB)],
              idx_v.at[1 - h], isem[1 - h])

        pltpu.async_copy(
            xp_hbm.at[idx_v.at[h, 0, 0]], rows_v.at[0], gsem[0])
        scats = [None] * NBUF
        for j in range(IB):
          b = j % NBUF
          nb = (j + 1) % NBUF
          pltpu.make_async_copy(
              xp_hbm.at[idx_v.at[h, j, 0]], rows_v.at[b], gsem[b]).wait()
          scats[b] = pltpu.async_copy(
              rows_v.at[b], acc_sh.at[idx_v.at[h, j, 1]], ssem[b], add=True)
          for i in range(CHUNK // 16):
            v = idx_v[h, j, 1, pl.ds(i * 16, 16)]
            plsc.addupdate_scatter(deg_v, [v >> 7, v & 127], ones16)
          if j + 1 < IB:
            if scats[nb] is not None:
              scats[nb].wait()
            pltpu.async_copy(
                xp_hbm.at[idx_v.at[h, j + 1, 0]], rows_v.at[nb], gsem[nb])
        for b in range(NBUF):
          if scats[b] is not None:
            scats[b].wait()

    pltpu.sync_copy(deg_v, deg_sh.at[ridx_v], add=True)
    plsc.subcore_barrier()

    pltpu.sync_copy(
        acc_sh.at[pl.ds(s * zrows, zrows)],
        out_hbm.at[c, pl.ds(s * zrows, zrows)],
    )

    @pl.when(s == 0)
    def _():
      pltpu.sync_copy(deg_sh, deg_hbm.at[c])

  return sc_agg


BN = 2528
NB = N_ACC // BN


def _tc_body(p_ref, degt_ref, x_ref, w_ref, b_ref, o_ref):
  i = pl.program_id(0)
  p = p_ref[0] + p_ref[1]
  degt = degt_ref[0] + degt_ref[1]
  sel = (lax.broadcasted_iota(jnp.int32, (NB, 1), 0) == i).astype(jnp.float32)
  dcol = jnp.dot(degt, sel, preferred_element_type=jnp.float32)
  agg = p / jnp.maximum(dcol, 1.0)
  h = jnp.dot(agg, w_ref[...], preferred_element_type=jnp.float32) + b_ref[...]
  o_ref[...] = jnp.maximum(h, 0.0) + x_ref[...]


def _tc_update(partials, degt, x, W, b2):
  return pl.pallas_call(
      _tc_body,
      grid=(NB,),
      in_specs=[
          pl.BlockSpec((NC, BN, D), lambda i: (0, i, 0)),
          pl.BlockSpec((NC, BN, NB), lambda i: (0, 0, 0)),
          pl.BlockSpec((BN, D), lambda i: (i, 0)),
          pl.BlockSpec((D, D), lambda i: (0, 0)),
          pl.BlockSpec((1, D), lambda i: (0, 0)),
      ],
      out_specs=pl.BlockSpec((BN, D), lambda i: (i, 0)),
      out_shape=jax.ShapeDtypeStruct((N, D), jnp.float32),
  )(partials, degt, x, W, b2)


@jax.jit
def kernel(x, edge_index, W, b):
  E = edge_index.shape[1]
  K = -(-E // (NW * CHUNK))
  K = -(-K // IB) * IB
  E_pad = NW * CHUNK * K
  pad = E_pad - E
  r = jnp.arange(pad, dtype=jnp.int32)
  src = jnp.concatenate([edge_index[0], r % N])
  dst = jnp.concatenate([edge_index[1], N + r % (N_ACC - N)])
  idx = jnp.stack([src.reshape(NW * K, CHUNK), dst.reshape(NW * K, CHUNK)],
                  axis=1)
  zeros = jnp.zeros((N_ACC, D), jnp.float32)
  partials, deg = _make_sc_agg(K)(x, idx, zeros)
  degt = jnp.transpose(
      deg.reshape(NC, ND * D)[:, :N_ACC].reshape(NC, NB, BN), (0, 2, 1))
  return _tc_update(partials, degt, x, W, b.reshape(1, D))


# speedup vs baseline: 1.3864x; 1.3864x over previous
import dataclasses
import functools

import jax
import jax.numpy as jnp
from jax import lax
from jax.experimental import pallas as pl
from jax.experimental.pallas import tpu as pltpu
from jax.experimental.pallas import tpu_sc as plsc

N = 10000
D = 128
N_ACC = 10112
NR = N_ACC // D
ND = 80
CHUNK = 128
NC = 2
NS = 16
NW = NC * NS
NBUF = 2
IB = 10


def _sc_compiler_params():
  cp = pltpu.CompilerParams(use_tc_tiling_on_sc=False)
  if "needs_layout_passes" in pltpu.CompilerParams.__dataclass_fields__:
    cp = dataclasses.replace(cp, needs_layout_passes=False)
  return cp


def _make_sc_agg(K):
  mesh = plsc.VectorSubcoreMesh(core_axis_name="c", subcore_axis_name="s")
  nblk = K // IB

  @functools.partial(
      pl.kernel,
      mesh=mesh,
      out_type=(
          jax.ShapeDtypeStruct((NC, N_ACC, D), jnp.bfloat16),
          jax.ShapeDtypeStruct((NC, ND, D), jnp.float32),
      ),
      scratch_types=[
          pltpu.VMEM_SHARED((N_ACC, D), jnp.bfloat16),
          pltpu.VMEM_SHARED((ND, D), jnp.float32),
          pltpu.VMEM((2, IB, 2, CHUNK), jnp.int32),
          pltpu.VMEM((NBUF, CHUNK, D), jnp.bfloat16),
          pltpu.VMEM((ND, D), jnp.float32),
          pltpu.VMEM((ND,), jnp.int32),
      ] + [pltpu.SemaphoreType.DMA] * (2 * NBUF + 2),
      compiler_params=_sc_compiler_params(),
  )
  def sc_agg(xp_hbm, idx_hbm, zeros_hbm, out_hbm, deg_hbm, acc_sh, deg_sh,
             idx_v, rows_v, deg_v, ridx_v, *sems):
    gsem = sems[:NBUF]
    ssem = sems[NBUF:2 * NBUF]
    isem = sems[2 * NBUF:]
    c = lax.axis_index("c")
    s = lax.axis_index("s")
    wid = c * NS + s
    zrows = N_ACC // NS

    pltpu.async_copy(idx_hbm.at[pl.ds(wid * K, IB)], idx_v.at[0], isem[0])

    pltpu.sync_copy(
        zeros_hbm.at[pl.ds(s * zrows, zrows)],
        acc_sh.at[pl.ds(s * zrows, zrows)],
    )

    ones16 = jnp.ones((16,), jnp.float32)
    zeros16 = jnp.zeros((16,), jnp.float32)

    @pl.loop(0, ND)
    def _(r):
      for i in range(D // 16):
        deg_v[r, pl.ds(i * 16, 16)] = zeros16

    for i in range(ND // 16):
      ridx_v[pl.ds(i * 16, 16)] = lax.iota(jnp.int32, 16) + 16 * i

    @pl.when(s == 0)
    def _():
      pltpu.sync_copy(deg_v, deg_sh)

    plsc.subcore_barrier()

    @pl.loop(0, nblk, step=2)
    def _(kb0):
      for h in range(2):
        kb = kb0 + h
        pltpu.make_async_copy(
            idx_hbm.at[pl.ds(wid * K + kb * IB, IB)], idx_v.at[h],
            isem[h]).wait()

        @pl.when(kb + 1 < nblk)
        def _():
          pltpu.async_copy(
              idx_hbm.at[pl.ds(wid * K + (kb + 1) * IB, IB)],
              idx_v.at[1 - h], isem[1 - h])

        pltpu.async_copy(
            xp_hbm.at[idx_v.at[h, 0, 0]], rows_v.at[0], gsem[0])
        scats = [None] * NBUF
        for j in range(IB):
          b = j % NBUF
          nb = (j + 1) % NBUF
          pltpu.make_async_copy(
              xp_hbm.at[idx_v.at[h, j, 0]], rows_v.at[b], gsem[b]).wait()
          scats[b] = pltpu.async_copy(
              rows_v.at[b], acc_sh.at[idx_v.at[h, j, 1]], ssem[b], add=True)
          for i in range(CHUNK // 16):
            v = idx_v[h, j, 1, pl.ds(i * 16, 16)]
            plsc.addupdate_scatter(deg_v, [v >> 7, v & 127], ones16)
          if j + 1 < IB:
            if scats[nb] is not None:
              scats[nb].wait()
            pltpu.async_copy(
                xp_hbm.at[idx_v.at[h, j + 1, 0]], rows_v.at[nb], gsem[nb])
        for b in range(NBUF):
          if scats[b] is not None:
            scats[b].wait()

    pltpu.sync_copy(deg_v, deg_sh.at[ridx_v], add=True)
    plsc.subcore_barrier()

    pltpu.sync_copy(
        acc_sh.at[pl.ds(s * zrows, zrows)],
        out_hbm.at[c, pl.ds(s * zrows, zrows)],
    )

    @pl.when(s == 0)
    def _():
      pltpu.sync_copy(deg_sh, deg_hbm.at[c])

  return sc_agg


BN = 2528
NB = N_ACC // BN


def _tc_body(p_ref, degt_ref, x_ref, w_ref, b_ref, o_ref):
  i = pl.program_id(0)
  p = (p_ref[0].astype(jnp.float32) + p_ref[1].astype(jnp.float32))
  degt = degt_ref[0] + degt_ref[1]
  sel = (lax.broadcasted_iota(jnp.int32, (NB, 1), 0) == i).astype(jnp.float32)
  dcol = jnp.dot(degt, sel, preferred_element_type=jnp.float32)
  agg = p / jnp.maximum(dcol, 1.0)
  h = jnp.dot(agg, w_ref[...], preferred_element_type=jnp.float32) + b_ref[...]
  o_ref[...] = jnp.maximum(h, 0.0) + x_ref[...]


def _tc_update(partials, degt, x, W, b2):
  return pl.pallas_call(
      _tc_body,
      grid=(NB,),
      in_specs=[
          pl.BlockSpec((NC, BN, D), lambda i: (0, i, 0)),
          pl.BlockSpec((NC, BN, NB), lambda i: (0, 0, 0)),
          pl.BlockSpec((BN, D), lambda i: (i, 0)),
          pl.BlockSpec((D, D), lambda i: (0, 0)),
          pl.BlockSpec((1, D), lambda i: (0, 0)),
      ],
      out_specs=pl.BlockSpec((BN, D), lambda i: (i, 0)),
      out_shape=jax.ShapeDtypeStruct((N, D), jnp.float32),
  )(partials, degt, x, W, b2)


@jax.jit
def kernel(x, edge_index, W, b):
  E = edge_index.shape[1]
  K = -(-E // (NW * CHUNK))
  K = -(-K // IB) * IB
  E_pad = NW * CHUNK * K
  pad = E_pad - E
  r = jnp.arange(pad, dtype=jnp.int32)
  src = jnp.concatenate([edge_index[0], r % N])
  dst = jnp.concatenate([edge_index[1], N + r % (N_ACC - N)])
  idx = jnp.stack([src.reshape(NW * K, CHUNK), dst.reshape(NW * K, CHUNK)],
                  axis=1)
  zeros = jnp.zeros((N_ACC, D), jnp.bfloat16)
  xh = x.astype(jnp.bfloat16)
  partials, deg = _make_sc_agg(K)(xh, idx, zeros)
  degt = jnp.transpose(
      deg.reshape(NC, ND * D)[:, :N_ACC].reshape(NC, NB, BN), (0, 2, 1))
  return _tc_update(partials, degt, x, W, b.reshape(1, D))
